# Initial kernel scaffold; baseline (speedup 1.0000x reference)
import functools

import jax
import jax.numpy as jnp
from jax import lax
from jax.experimental import pallas as pl
from jax.experimental.pallas import tpu as pltpu
from jax.experimental.pallas import tpu_sc as plsc

N_NODES = 10000
N_EDGES = 320000
D = 128

NC = 2
NS = 16
NW = NC * NS

T = 128
EP = 327680
EPT = EP // NW
NT = EPT // T
NPAD = 10016
ROWS_PER_TILE = NPAD // NS


def _sc_aggregate(src, dst, x, zeros):
    mesh = plsc.VectorSubcoreMesh(core_axis_name="c", subcore_axis_name="s")

    @functools.partial(
        pl.kernel,
        mesh=mesh,
        out_type=jax.ShapeDtypeStruct((NC, NPAD, D), jnp.float32),
        scratch_types=[
            pltpu.VMEM((2, T), jnp.int32),
            pltpu.VMEM((2, T), jnp.int32),
            pltpu.VMEM((2, T, D), jnp.float32),
            pltpu.VMEM_SHARED((NPAD, D), jnp.float32),
            pltpu.SemaphoreType.DMA,
            pltpu.SemaphoreType.DMA,
        ],
    )
    def agg(src_hbm, dst_hbm, x_hbm, z_hbm, out_hbm, sidx, didx, rows, aggr,
            sem0, sem1):
        c = lax.axis_index("c")
        s = lax.axis_index("s")
        wid = s * NC + c
        base = wid * EPT
        sems = (sem0, sem1)

        @pl.when(s == 0)
        def _():
            pltpu.sync_copy(z_hbm, aggr)

        plsc.subcore_barrier()

        def fill(b, t):
            off = base + t * T
            pltpu.sync_copy(src_hbm.at[pl.ds(off, T)], sidx.at[b])
            pltpu.sync_copy(dst_hbm.at[pl.ds(off, T)], didx.at[b])
            pltpu.async_copy(x_hbm.at[sidx.at[b]], rows.at[b], sems[b])

        def drain_and_scatter(b):
            pltpu.make_async_copy(x_hbm.at[sidx.at[b]], rows.at[b],
                                  sems[b]).wait()
            pltpu.sync_copy(rows.at[b], aggr.at[didx.at[b]], add=True)

        for b in range(2):
            fill(b, b)

        def body(i, carry):
            t0 = i * 2
            for b in range(2):
                drain_and_scatter(b)
                fill(b, t0 + 2 + b)
            return carry

        lax.fori_loop(0, NT // 2 - 1, body, 0)
        for b in range(2):
            drain_and_scatter(b)

        plsc.subcore_barrier()
        r0 = s * ROWS_PER_TILE
        pltpu.sync_copy(aggr.at[pl.ds(r0, ROWS_PER_TILE)],
                        out_hbm.at[c].at[pl.ds(r0, ROWS_PER_TILE)])

    return agg(src, dst, x, zeros)


BLK = 400


def _mlp_body(x_ref, p_ref, w1_ref, b1_ref, w2_ref, b2_ref, g_ref, be_ref,
              wr_ref, br_ref, o_ref):
    xb = x_ref[...]
    h = xb + p_ref[0] + p_ref[1]
    h = lax.dot_general(h, w1_ref[...], (((1,), (1,)), ((), ())),
                        preferred_element_type=jnp.float32,
                        precision=lax.Precision.HIGHEST)
    h = jnp.maximum(h + b1_ref[...], 0.0)
    h = lax.dot_general(h, w2_ref[...], (((1,), (1,)), ((), ())),
                        preferred_element_type=jnp.float32,
                        precision=lax.Precision.HIGHEST) + b2_ref[...]
    mean = jnp.mean(h, axis=1, keepdims=True)
    hc = h - mean
    var = jnp.mean(hc * hc, axis=1, keepdims=True)
    hn = hc * lax.rsqrt(var + 1e-5) * g_ref[...] + be_ref[...]
    res = lax.dot_general(xb, wr_ref[...], (((1,), (1,)), ((), ())),
                          preferred_element_type=jnp.float32,
                          precision=lax.Precision.HIGHEST) + br_ref[...]
    o_ref[...] = hn + res


def _tc_mlp(x, partials, W1, b1, W2, b2, gamma, beta, Wres, bres):
    grid = (N_NODES // BLK,)
    full = lambda shape: pl.BlockSpec(shape, lambda i: (0,) * len(shape))
    return pl.pallas_call(
        _mlp_body,
        grid=grid,
        in_specs=[
            pl.BlockSpec((BLK, D), lambda i: (i, 0)),
            pl.BlockSpec((NC, BLK, D), lambda i: (0, i, 0)),
            full((D, D)), full((1, D)),
            full((D, D)), full((1, D)),
            full((1, D)), full((1, D)),
            full((D, D)), full((1, D)),
        ],
        out_specs=pl.BlockSpec((BLK, D), lambda i: (i, 0)),
        out_shape=jax.ShapeDtypeStruct((N_NODES, D), jnp.float32),
    )(x, partials, W1, b1, W2, b2, gamma, beta, Wres, bres)


def kernel(x, edge_index, W1, b1, W2, b2, gamma, beta, Wres, bres):
    src = edge_index[0].astype(jnp.int32)
    dst = edge_index[1].astype(jnp.int32)
    pad = EP - N_EDGES
    src = jnp.concatenate([src, jnp.zeros((pad,), jnp.int32)])
    dst = jnp.concatenate([dst, jnp.full((pad,), N_NODES, jnp.int32)])
    zeros = jnp.zeros((NPAD, D), jnp.float32)
    partials = _sc_aggregate(src, dst, x, zeros)
    row = lambda v: v.reshape(1, D)
    return _tc_mlp(x, partials, W1, row(b1), W2, row(b2), row(gamma),
                   row(beta), Wres, row(bres))


# speedup vs baseline: 3.6104x; 3.6104x over previous
import functools

import jax
import jax.numpy as jnp
from jax import lax
from jax.experimental import pallas as pl
from jax.experimental.pallas import tpu as pltpu
from jax.experimental.pallas import tpu_sc as plsc

N_NODES = 10000
N_EDGES = 320000
D = 128

NC = 2
NS = 16
NW = NC * NS

T = 128
EP = 327680
EPT = EP // NW
NT = EPT // T
NPAD = 10112
ROWS_PER_TILE = NPAD // NS


def _sc_aggregate(src, dst, x, zeros):
    mesh = plsc.VectorSubcoreMesh(core_axis_name="c", subcore_axis_name="s")

    @functools.partial(
        pl.kernel,
        mesh=mesh,
        out_type=jax.ShapeDtypeStruct((NC, NPAD, D), jnp.float32),
        scratch_types=[
            pltpu.VMEM((2, T), jnp.int32),
            pltpu.VMEM((2, T), jnp.int32),
            pltpu.VMEM((2, T, D), jnp.float32),
            pltpu.VMEM_SHARED((NPAD, D), jnp.float32),
            pltpu.SemaphoreType.DMA,
            pltpu.SemaphoreType.DMA,
        ],
    )
    def agg(src_hbm, dst_hbm, x_hbm, z_hbm, out_hbm, sidx, didx, rows, aggr,
            sem0, sem1):
        c = lax.axis_index("c")
        s = lax.axis_index("s")
        wid = s * NC + c
        base = wid * EPT
        sems = (sem0, sem1)

        @pl.when(s == 0)
        def _():
            pltpu.sync_copy(z_hbm, aggr)

        plsc.subcore_barrier()

        def fill(b, t):
            off = base + t * T
            pltpu.sync_copy(src_hbm.at[pl.ds(off, T)], sidx.at[b])
            pltpu.sync_copy(dst_hbm.at[pl.ds(off, T)], didx.at[b])
            pltpu.async_copy(x_hbm.at[sidx.at[b]], rows.at[b], sems[b])

        def drain_and_scatter(b):
            pltpu.make_async_copy(x_hbm.at[sidx.at[b]], rows.at[b],
                                  sems[b]).wait()
            pltpu.sync_copy(rows.at[b], aggr.at[didx.at[b]], add=True)

        for b in range(2):
            fill(b, b)

        def body(i, carry):
            t0 = i * 2
            for b in range(2):
                drain_and_scatter(b)
                fill(b, t0 + 2 + b)
            return carry

        lax.fori_loop(0, NT // 2 - 1, body, 0)
        for b in range(2):
            drain_and_scatter(b)

        plsc.subcore_barrier()
        r0 = s * ROWS_PER_TILE
        pltpu.sync_copy(aggr.at[pl.ds(r0, ROWS_PER_TILE)],
                        out_hbm.at[c].at[pl.ds(r0, ROWS_PER_TILE)])

    return agg(src, dst, x, zeros)


BLK = 400


def _mlp_body(x_ref, p_ref, w1_ref, b1_ref, w2_ref, b2_ref, g_ref, be_ref,
              wr_ref, br_ref, o_ref):
    xb = x_ref[...]
    h = xb + p_ref[0] + p_ref[1]
    h = lax.dot_general(h, w1_ref[...], (((1,), (1,)), ((), ())),
                        preferred_element_type=jnp.float32,
                        precision=lax.Precision.HIGHEST)
    h = jnp.maximum(h + b1_ref[...], 0.0)
    h = lax.dot_general(h, w2_ref[...], (((1,), (1,)), ((), ())),
                        preferred_element_type=jnp.float32,
                        precision=lax.Precision.HIGHEST) + b2_ref[...]
    mean = jnp.mean(h, axis=1, keepdims=True)
    hc = h - mean
    var = jnp.mean(hc * hc, axis=1, keepdims=True)
    hn = hc * lax.rsqrt(var + 1e-5) * g_ref[...] + be_ref[...]
    res = lax.dot_general(xb, wr_ref[...], (((1,), (1,)), ((), ())),
                          preferred_element_type=jnp.float32,
                          precision=lax.Precision.HIGHEST) + br_ref[...]
    o_ref[...] = hn + res


def _tc_mlp(x, partials, W1, b1, W2, b2, gamma, beta, Wres, bres):
    grid = (N_NODES // BLK,)
    full = lambda shape: pl.BlockSpec(shape, lambda i: (0,) * len(shape))
    return pl.pallas_call(
        _mlp_body,
        grid=grid,
        in_specs=[
            pl.BlockSpec((BLK, D), lambda i: (i, 0)),
            pl.BlockSpec((NC, BLK, D), lambda i: (0, i, 0)),
            full((D, D)), full((1, D)),
            full((D, D)), full((1, D)),
            full((1, D)), full((1, D)),
            full((D, D)), full((1, D)),
        ],
        out_specs=pl.BlockSpec((BLK, D), lambda i: (i, 0)),
        out_shape=jax.ShapeDtypeStruct((N_NODES, D), jnp.float32),
    )(x, partials, W1, b1, W2, b2, gamma, beta, Wres, bres)


def kernel(x, edge_index, W1, b1, W2, b2, gamma, beta, Wres, bres):
    src = edge_index[0].astype(jnp.int32)
    dst = edge_index[1].astype(jnp.int32)
    pad = EP - N_EDGES
    src = jnp.concatenate([src, jnp.zeros((pad,), jnp.int32)])
    dst = jnp.concatenate([dst, jnp.full((pad,), N_NODES, jnp.int32)])
    zeros = jnp.zeros((NPAD, D), jnp.float32)
    partials = _sc_aggregate(src, dst, x, zeros)
    row = lambda v: v.reshape(1, D)
    return _tc_mlp(x, partials, W1, row(b1), W2, row(b2), row(gamma),
                   row(beta), Wres, row(bres))
